# Initial kernel scaffold; baseline (speedup 1.0000x reference)
import jax
import jax.numpy as jnp
from jax.experimental import pallas as pl


def kernel(acoustic_input, text_input, speaker_input, embedding_table, speaker_table, W, b):
    raise NotImplementedError("write your pallas kernel here")


# speedup vs baseline: 3.5544x; 3.5544x over previous
import functools

import jax
import jax.numpy as jnp
from jax import lax
from jax.experimental import pallas as pl
from jax.experimental.pallas import tpu as pltpu
from jax.experimental.pallas import tpu_sc as plsc

B, L = 4096, 200
TEXT_DIM, AUDIO_DIM, SPKR_DIM = 128, 64, 32
VOCAB, N_SPK = 1000000, 1000

NW = 32
RW = B // NW
NG = RW // 16

_EROWS = 4000


def _eproj_body(tab_ref, w_ref, out_ref):
    out_ref[...] = lax.dot_general(
        w_ref[...], tab_ref[...],
        dimension_numbers=(((1,), (1,)), ((), ())),
        preferred_element_type=jnp.float32)[None]


def _eproj(table, w_e):
    nblk = VOCAB // _EROWS
    return pl.pallas_call(
        _eproj_body,
        grid=(nblk,),
        in_specs=[
            pl.BlockSpec((_EROWS, TEXT_DIM), lambda i: (i, 0)),
            pl.BlockSpec((1, TEXT_DIM), lambda i: (0, 0)),
        ],
        out_specs=pl.BlockSpec((1, 1, _EROWS), lambda i: (i, 0, 0)),
        out_shape=jax.ShapeDtypeStruct((nblk, 1, _EROWS), jnp.float32),
    )(table, w_e)


_AROWS = 64


def _aacc_body(ac_ref, w_ref, b_ref, out_ref):
    out_ref[...] = (lax.dot_general(
        w_ref[...], ac_ref[...],
        dimension_numbers=(((1,), (1,)), ((), ())),
        preferred_element_type=jnp.float32) + b_ref[0, 0] * float(L))[None]


def _aacc(ac2d, w_tile, b2d):
    nblk = B // _AROWS
    k = L * AUDIO_DIM
    return pl.pallas_call(
        _aacc_body,
        grid=(nblk,),
        in_specs=[
            pl.BlockSpec((_AROWS, k), lambda i: (i, 0)),
            pl.BlockSpec((1, k), lambda i: (0, 0)),
            pl.BlockSpec((1, 1), lambda i: (0, 0)),
        ],
        out_specs=pl.BlockSpec((1, 1, _AROWS), lambda i: (i, 0, 0)),
        out_shape=jax.ShapeDtypeStruct((nblk, 1, _AROWS), jnp.float32),
    )(ac2d, w_tile, b2d)


def _sproj_body(tab_ref, w_ref, out_ref):
    out_ref[...] = lax.dot_general(
        w_ref[...], tab_ref[...],
        dimension_numbers=(((1,), (1,)), ((), ())),
        preferred_element_type=jnp.float32)


def _sproj(table, w_s):
    return pl.pallas_call(
        _sproj_body,
        in_specs=[
            pl.BlockSpec((N_SPK, SPKR_DIM), lambda: (0, 0)),
            pl.BlockSpec((1, SPKR_DIM), lambda: (0, 0)),
        ],
        out_specs=pl.BlockSpec((1, N_SPK), lambda: (0, 0)),
        out_shape=jax.ShapeDtypeStruct((1, N_SPK), jnp.float32),
    )(table, w_s)


def _sc_body(eproj_hbm, text_hbm, spk_hbm, aacc_hbm, sproj_hbm, out_hbm,
             idx_t, idx_s, vals, svals, a_v, out_v, sem):
    wid = lax.axis_index("s") * 2 + lax.axis_index("c")

    pltpu.sync_copy(text_hbm.at[wid], idx_t)
    pltpu.sync_copy(spk_hbm.at[wid], idx_s)
    pltpu.sync_copy(aacc_hbm.at[pl.ds(wid * RW, RW)], a_v)
    ec = pltpu.async_copy(eproj_hbm.at[idx_t], vals, sem)
    sc = pltpu.async_copy(sproj_hbm.at[idx_s], svals, sem)
    ec.wait()
    sc.wait()

    def body(t, accs):
        new = []
        for k in range(NG):
            sl = pl.ds(t * RW + k * 16, 16)
            new.append(accs[k] + vals[sl] + svals[sl])
        return tuple(new)

    accs = lax.fori_loop(
        0, L, body, tuple(jnp.zeros((16,), jnp.float32) for _ in range(NG)))

    for k in range(NG):
        sl = pl.ds(k * 16, 16)
        x = (accs[k] + a_v[sl]) * (1.0 / float(L))
        out_v[sl] = 1.0 / (1.0 + jnp.exp(-x))

    pltpu.sync_copy(out_v, out_hbm.at[pl.ds(wid * RW, RW)])


@functools.partial(
    pl.kernel,
    out_type=jax.ShapeDtypeStruct((B,), jnp.float32),
    mesh=plsc.VectorSubcoreMesh(core_axis_name="c", subcore_axis_name="s"),
    scratch_types=[
        pltpu.VMEM((L * RW,), jnp.int32),
        pltpu.VMEM((L * RW,), jnp.int32),
        pltpu.VMEM((L * RW,), jnp.float32),
        pltpu.VMEM((L * RW,), jnp.float32),
        pltpu.VMEM((RW,), jnp.float32),
        pltpu.VMEM((RW,), jnp.float32),
        pltpu.SemaphoreType.DMA,
    ],
)
def _sc_combine(eproj_hbm, text_hbm, spk_hbm, aacc_hbm, sproj_hbm, out_hbm,
                idx_t, idx_s, vals, svals, a_v, out_v, sem):
    _sc_body(eproj_hbm, text_hbm, spk_hbm, aacc_hbm, sproj_hbm, out_hbm,
             idx_t, idx_s, vals, svals, a_v, out_v, sem)


def kernel(acoustic_input, text_input, speaker_input, embedding_table,
           speaker_table, W, b):
    w_a = W[:, :AUDIO_DIM]
    w_e = W[:, AUDIO_DIM:AUDIO_DIM + TEXT_DIM]
    w_s = W[:, AUDIO_DIM + TEXT_DIM:]

    eproj = _eproj(embedding_table, w_e).reshape(VOCAB)
    sproj = _sproj(speaker_table, w_s).reshape(N_SPK)

    ac2d = acoustic_input.reshape(B, L * AUDIO_DIM)
    w_tile = jnp.tile(w_a, (1, L))
    aacc = _aacc(ac2d, w_tile, b.reshape(1, 1)).reshape(B)

    text_t = text_input.T.reshape(L, NW, RW).transpose(1, 0, 2).reshape(NW, L * RW)
    spk_t = speaker_input.T.reshape(L, NW, RW).transpose(1, 0, 2).reshape(NW, L * RW)

    return _sc_combine(eproj, text_t, spk_t, aacc, sproj)


# speedup vs baseline: 5.2319x; 1.4720x over previous
import functools

import jax
import jax.numpy as jnp
from jax import lax
from jax.experimental import pallas as pl
from jax.experimental.pallas import tpu as pltpu
from jax.experimental.pallas import tpu_sc as plsc

B, L = 4096, 200
TEXT_DIM, AUDIO_DIM, SPKR_DIM = 128, 64, 32
VOCAB, N_SPK = 1000000, 1000

NW = 32
RW = B // NW
NG = RW // 16

_EROWS = 4000


def _eproj_body(tab_ref, w_ref, out_ref):
    out_ref[...] = lax.dot_general(
        w_ref[...], tab_ref[...],
        dimension_numbers=(((1,), (1,)), ((), ())),
        preferred_element_type=jnp.float32)[None]


def _eproj(table, w_e):
    nblk = VOCAB // _EROWS
    return pl.pallas_call(
        _eproj_body,
        grid=(nblk,),
        in_specs=[
            pl.BlockSpec((_EROWS, TEXT_DIM), lambda i: (i, 0)),
            pl.BlockSpec((1, TEXT_DIM), lambda i: (0, 0)),
        ],
        out_specs=pl.BlockSpec((1, 1, _EROWS), lambda i: (i, 0, 0)),
        out_shape=jax.ShapeDtypeStruct((nblk, 1, _EROWS), jnp.float32),
    )(table, w_e)


_AROWS = 128


def _aacc_body(ac_ref, w_ref, b_ref, out_ref):
    pooled = jnp.sum(ac_ref[...], axis=1)
    out_ref[...] = (lax.dot_general(
        w_ref[...], pooled,
        dimension_numbers=(((1,), (1,)), ((), ())),
        preferred_element_type=jnp.float32) + b_ref[0, 0] * float(L))[None]


def _aacc(acoustic, w_a, b2d):
    nblk = B // _AROWS
    return pl.pallas_call(
        _aacc_body,
        grid=(nblk,),
        in_specs=[
            pl.BlockSpec((_AROWS, L, AUDIO_DIM), lambda i: (i, 0, 0)),
            pl.BlockSpec((1, AUDIO_DIM), lambda i: (0, 0)),
            pl.BlockSpec((1, 1), lambda i: (0, 0)),
        ],
        out_specs=pl.BlockSpec((1, 1, _AROWS), lambda i: (i, 0, 0)),
        out_shape=jax.ShapeDtypeStruct((nblk, 1, _AROWS), jnp.float32),
    )(acoustic, w_a, b2d)


def _sproj_body(tab_ref, w_ref, out_ref):
    out_ref[...] = lax.dot_general(
        w_ref[...], tab_ref[...],
        dimension_numbers=(((1,), (1,)), ((), ())),
        preferred_element_type=jnp.float32)


def _sproj(table, w_s):
    return pl.pallas_call(
        _sproj_body,
        in_specs=[
            pl.BlockSpec((N_SPK, SPKR_DIM), lambda: (0, 0)),
            pl.BlockSpec((1, SPKR_DIM), lambda: (0, 0)),
        ],
        out_specs=pl.BlockSpec((1, N_SPK), lambda: (0, 0)),
        out_shape=jax.ShapeDtypeStruct((1, N_SPK), jnp.float32),
    )(table, w_s)


_SCH = 10000
_NST = VOCAB // _SCH
_NCH = 2
_CW = L * RW // _NCH
_LT = L // _NCH


def _sc_body(eproj_hbm, text_hbm, spk_hbm, aacc_hbm, sproj_hbm, out_hbm,
             idx_t, idx_s, vals, svals, a_v, out_v, eproj_sh, sproj_sh, sem):
    cid = lax.axis_index("c")
    sid = lax.axis_index("s")
    wid = sid * 2 + cid

    for j in range((_NST + 15) // 16):
        c = sid + j * 16

        @pl.when(c < _NST)
        def _stage():
            off = c * _SCH
            pltpu.sync_copy(eproj_hbm.at[pl.ds(off, _SCH)],
                            vals.at[pl.ds(0, _SCH)])
            pltpu.sync_copy(vals.at[pl.ds(0, _SCH)],
                            eproj_sh.at[pl.ds(off, _SCH)])

    @pl.when(sid == 15)
    def _stage_s():
        pltpu.sync_copy(sproj_hbm, svals.at[pl.ds(0, N_SPK)])
        pltpu.sync_copy(svals.at[pl.ds(0, N_SPK)], sproj_sh)

    pltpu.sync_copy(aacc_hbm.at[pl.ds(wid * RW, RW)], a_v)
    plsc.subcore_barrier()

    accs = tuple(jnp.zeros((16,), jnp.float32) for _ in range(NG))
    for ch in range(_NCH):
        pltpu.sync_copy(text_hbm.at[wid, ch], idx_t)
        pltpu.sync_copy(spk_hbm.at[wid, ch], idx_s)
        ec = pltpu.async_copy(eproj_sh.at[idx_t], vals, sem)
        sc = pltpu.async_copy(sproj_sh.at[idx_s], svals, sem)
        ec.wait()
        sc.wait()

        def body(t, accs):
            new = []
            for k in range(NG):
                sl = pl.ds(t * RW + k * 16, 16)
                new.append(accs[k] + vals[sl] + svals[sl])
            return tuple(new)

        accs = lax.fori_loop(0, _LT, body, accs)

    for k in range(NG):
        sl = pl.ds(k * 16, 16)
        x = (accs[k] + a_v[sl]) * (1.0 / float(L))
        out_v[sl] = 1.0 / (1.0 + jnp.exp(-x))

    pltpu.sync_copy(out_v, out_hbm.at[pl.ds(wid * RW, RW)])


@functools.partial(
    pl.kernel,
    out_type=jax.ShapeDtypeStruct((B,), jnp.float32),
    mesh=plsc.VectorSubcoreMesh(core_axis_name="c", subcore_axis_name="s"),
    scratch_types=[
        pltpu.VMEM((_CW,), jnp.int32),
        pltpu.VMEM((_CW,), jnp.int32),
        pltpu.VMEM((_CW,), jnp.float32),
        pltpu.VMEM((_CW,), jnp.float32),
        pltpu.VMEM((RW,), jnp.float32),
        pltpu.VMEM((RW,), jnp.float32),
        pltpu.VMEM_SHARED((VOCAB,), jnp.float32),
        pltpu.VMEM_SHARED((N_SPK,), jnp.float32),
        pltpu.SemaphoreType.DMA,
    ],
)
def _sc_combine(eproj_hbm, text_hbm, spk_hbm, aacc_hbm, sproj_hbm, out_hbm,
                idx_t, idx_s, vals, svals, a_v, out_v, eproj_sh, sproj_sh, sem):
    _sc_body(eproj_hbm, text_hbm, spk_hbm, aacc_hbm, sproj_hbm, out_hbm,
             idx_t, idx_s, vals, svals, a_v, out_v, eproj_sh, sproj_sh, sem)


def kernel(acoustic_input, text_input, speaker_input, embedding_table,
           speaker_table, W, b):
    w_a = W[:, :AUDIO_DIM]
    w_e = W[:, AUDIO_DIM:AUDIO_DIM + TEXT_DIM]
    w_s = W[:, AUDIO_DIM + TEXT_DIM:]

    eproj = _eproj(embedding_table, w_e).reshape(VOCAB)
    sproj = _sproj(speaker_table, w_s).reshape(N_SPK)

    aacc = _aacc(acoustic_input, w_a, b.reshape(1, 1)).reshape(B)

    text_t = text_input.T.reshape(L, NW, RW).transpose(1, 0, 2).reshape(NW, _NCH, _CW)
    spk_t = speaker_input.T.reshape(L, NW, RW).transpose(1, 0, 2).reshape(NW, _NCH, _CW)

    return _sc_combine(eproj, text_t, spk_t, aacc, sproj)
